# baseline (speedup 1.0000x reference)
import functools

import jax
import jax.numpy as jnp
from jax import lax
from jax.experimental import pallas as pl
from jax.experimental.pallas import tpu as pltpu
from jax.experimental.pallas import tpu_sc as plsc

N = 16384
D = 128
L = 16
NC, NS = 2, 16
NW = NC * NS
ROWS_PER_W = N // NW
VPR = D // L
CH = 128
NCH = ROWS_PER_W // CH
NBUF = 3

_mesh = plsc.VectorSubcoreMesh(core_axis_name="c", subcore_axis_name="s")


def _permute(v, idx):
    dnums = lax.GatherDimensionNumbers(
        offset_dims=(), collapsed_slice_dims=(0,), start_index_map=(0,))
    return lax.gather(v, idx[:, None], dnums, (1,),
                      mode=lax.GatherScatterMode.PROMISE_IN_BOUNDS)


@functools.partial(
    pl.kernel,
    mesh=_mesh,
    out_type=jax.ShapeDtypeStruct((N, D), jnp.float32),
    scratch_types=[
        pltpu.VMEM((NBUF, CH, D), jnp.float32),
        pltpu.VMEM((ROWS_PER_W,), jnp.int32),
        pltpu.VMEM((L,), jnp.float32),
        pltpu.VMEM((L,), jnp.float32),
        pltpu.SemaphoreType.DMA((NBUF,)),
        pltpu.SemaphoreType.DMA((NBUF,)),
    ],
)
def _affine_sc(z_hbm, obs_hbm, coefs_hbm, offsets_hbm, out_hbm,
               zbuf, obsbuf, cbuf, obuf, in_sem, out_sem):
    wid = lax.axis_index("s") * NC + lax.axis_index("c")
    base = wid * ROWS_PER_W

    pltpu.sync_copy(coefs_hbm, cbuf.at[pl.ds(0, 8)])
    pltpu.sync_copy(offsets_hbm, obuf.at[pl.ds(0, 8)])
    pltpu.sync_copy(obs_hbm.at[pl.ds(base, ROWS_PER_W)], obsbuf)

    ctab = cbuf[...]
    otab = obuf[...]

    def start_in(i):
        return pltpu.async_copy(
            z_hbm.at[pl.ds(base + i * CH, CH)], zbuf.at[i % NBUF],
            in_sem.at[i % NBUF])

    def start_out(i):
        return pltpu.async_copy(
            zbuf.at[i % NBUF], out_hbm.at[pl.ds(base + i * CH, CH)],
            out_sem.at[i % NBUF])

    def compute(i):
        b = i % NBUF

        @plsc.parallel_loop(0, CH // L, 1)
        def group_body(t):
            r0 = t * L
            idx16 = obsbuf[pl.ds(i * CH + r0, L)]
            c16 = _permute(ctab, idx16)
            o16 = _permute(otab, idx16)
            for k in range(L):
                lane = jnp.full((L,), k, dtype=jnp.int32)
                c = _permute(c16, lane)
                o = _permute(o16, lane)
                for j in range(VPR):
                    s = pl.ds(j * L, L)
                    zbuf[b, r0 + k, s] = c * zbuf[b, r0 + k, s] + o

    in_h = [None] * NCH
    out_h = [None] * NCH
    in_h[0] = start_in(0)
    if NCH > 1:
        in_h[1] = start_in(1)
    for i in range(NCH):
        in_h[i].wait()
        compute(i)
        out_h[i] = start_out(i)
        nxt = i + 2
        if nxt < NCH:
            if nxt - NBUF >= 0:
                out_h[nxt - NBUF].wait()
            in_h[nxt] = start_in(nxt)
    for i in range(max(0, NCH - NBUF), NCH):
        out_h[i].wait()


def kernel(z, obs, coefs, offsets):
    return _affine_sc(z, obs.astype(jnp.int32), coefs, offsets)


# speedup vs baseline: 2.1109x; 2.1109x over previous
import functools

import jax
import jax.numpy as jnp
from jax import lax
from jax.experimental import pallas as pl
from jax.experimental.pallas import tpu as pltpu
from jax.experimental.pallas import tpu_sc as plsc

N = 16384
D = 128
L = 16
NC, NS = 2, 16
NW = NC * NS
ROWS_PER_W = N // NW
VPR = D // L
CH = 128
NCH = ROWS_PER_W // CH
NBUF = 3

_mesh = plsc.VectorSubcoreMesh(core_axis_name="c", subcore_axis_name="s")


def _permute(v, idx):
    dnums = lax.GatherDimensionNumbers(
        offset_dims=(), collapsed_slice_dims=(0,), start_index_map=(0,))
    return lax.gather(v, idx[:, None], dnums, (1,),
                      mode=lax.GatherScatterMode.PROMISE_IN_BOUNDS)


@functools.partial(
    pl.kernel,
    mesh=_mesh,
    out_type=jax.ShapeDtypeStruct((N, D), jnp.float32),
    scratch_types=[
        pltpu.VMEM((NBUF, CH, D), jnp.float32),
        pltpu.VMEM((ROWS_PER_W,), jnp.int32),
        pltpu.VMEM((L,), jnp.float32),
        pltpu.VMEM((L,), jnp.float32),
        pltpu.SemaphoreType.DMA((NBUF,)),
        pltpu.SemaphoreType.DMA((NBUF,)),
    ],
)
def _affine_sc(z_hbm, obs_hbm, coefs_hbm, offsets_hbm, out_hbm,
               zbuf, obsbuf, cbuf, obuf, in_sem, out_sem):
    wid = lax.axis_index("s") * NC + lax.axis_index("c")
    base = wid * ROWS_PER_W

    pltpu.sync_copy(coefs_hbm, cbuf.at[pl.ds(0, 8)])
    pltpu.sync_copy(offsets_hbm, obuf.at[pl.ds(0, 8)])
    pltpu.sync_copy(obs_hbm.at[pl.ds(base, ROWS_PER_W)], obsbuf)

    ctab = cbuf[...]
    otab = obuf[...]

    def start_in(i):
        return pltpu.async_copy(
            z_hbm.at[pl.ds(base + i * CH, CH)], zbuf.at[i % NBUF],
            in_sem.at[i % NBUF])

    def start_out(i):
        return pltpu.async_copy(
            zbuf.at[i % NBUF], out_hbm.at[pl.ds(base + i * CH, CH)],
            out_sem.at[i % NBUF])

    def compute(i):
        b = i % NBUF

        @plsc.parallel_loop(0, CH // L, 1)
        def group_body(t):
            r0 = t * L
            idx16 = obsbuf[pl.ds(i * CH + r0, L)]
            c16 = _permute(ctab, idx16)
            o16 = _permute(otab, idx16)
            for k in range(L):
                lane = jnp.full((L,), k, dtype=jnp.int32)
                c = _permute(c16, lane)
                o = _permute(o16, lane)
                for j in range(VPR):
                    s = pl.ds(j * L, L)
                    zbuf[b, r0 + k, s] = c * zbuf[b, r0 + k, s] + o

    in_h = [None] * NCH
    out_h = [None] * NCH
    in_h[0] = start_in(0)
    if NCH > 1:
        in_h[1] = start_in(1)
    for i in range(NCH):
        in_h[i].wait()
        compute(i)
        out_h[i] = start_out(i)
        nxt = i + 2
        if nxt < NCH:
            if nxt - NBUF >= 0:
                out_h[nxt - NBUF].wait()
            in_h[nxt] = start_in(nxt)
    for i in range(max(0, NCH - NBUF), NCH):
        out_h[i].wait()


TC_BLK = 1024
TC_NB = N // TC_BLK


def _affine_tc_body(obs_ref, coefs_ref, offsets_ref, z_ref, o_ref):
    ob = obs_ref[0, 0, :]
    c = jnp.zeros((TC_BLK,), jnp.float32)
    o = jnp.zeros((TC_BLK,), jnp.float32)
    for k in range(8):
        sel = ob == k
        c = jnp.where(sel, coefs_ref[k], c)
        o = jnp.where(sel, offsets_ref[k], o)
    o_ref[...] = c[:, None] * z_ref[...] + o[:, None]


def _affine_tc(z, obs, coefs, offsets):
    obs3 = obs.reshape(TC_NB, 1, TC_BLK)
    return pl.pallas_call(
        _affine_tc_body,
        grid=(TC_NB,),
        in_specs=[
            pl.BlockSpec((1, 1, TC_BLK), lambda i: (i, 0, 0)),
            pl.BlockSpec(memory_space=pltpu.SMEM),
            pl.BlockSpec(memory_space=pltpu.SMEM),
            pl.BlockSpec((TC_BLK, D), lambda i: (i, 0)),
        ],
        out_specs=pl.BlockSpec((TC_BLK, D), lambda i: (i, 0)),
        out_shape=jax.ShapeDtypeStruct((N, D), jnp.float32),
    )(obs3, coefs, offsets, z)


def kernel(z, obs, coefs, offsets):
    return _affine_tc(z, obs.astype(jnp.int32), coefs, offsets)
